# baseline (speedup 1.0000x reference)
import functools

import jax
import jax.numpy as jnp
from jax import lax
from jax.experimental import pallas as pl
from jax.experimental.pallas import tpu as pltpu
from jax.experimental.pallas import tpu_sc as plsc

NUM_CORES = 2
NUM_SUBCORES = 16
NUM_WORKERS = NUM_CORES * NUM_SUBCORES
LANES = 16
PAD_D = 8


@jax.jit
def _score_pairs(user_idx, item_idx, user_rows8, item_rows8):
    batch = user_idx.shape[0]
    b_per_w = batch // NUM_WORKERS
    n_groups = b_per_w // LANES

    mesh = plsc.VectorSubcoreMesh(
        core_axis_name="c", subcore_axis_name="s",
        num_cores=NUM_CORES, num_subcores=NUM_SUBCORES)

    @functools.partial(
        pl.kernel,
        out_type=jax.ShapeDtypeStruct((batch,), jnp.float32),
        mesh=mesh,
        compiler_params=pltpu.CompilerParams(
            needs_layout_passes=False, use_tc_tiling_on_sc=False),
        scratch_types=[
            pltpu.VMEM((b_per_w,), jnp.int32),
            pltpu.VMEM((b_per_w,), jnp.int32),
            pltpu.VMEM((b_per_w, PAD_D), jnp.float32),
            pltpu.VMEM((b_per_w, PAD_D), jnp.float32),
            pltpu.VMEM((b_per_w,), jnp.float32),
            pltpu.SemaphoreType.DMA,
        ],
    )
    def run(ui_hbm, ii_hbm, ut_hbm, it_hbm, out_hbm,
            idx_u, idx_i, rows_u, rows_i, out_v, sem):
        wid = lax.axis_index("s") * NUM_CORES + lax.axis_index("c")
        base = wid * b_per_w

        pltpu.sync_copy(ui_hbm.at[pl.ds(base, b_per_w)], idx_u)
        pltpu.sync_copy(ii_hbm.at[pl.ds(base, b_per_w)], idx_i)

        cp_u = pltpu.async_copy(ut_hbm.at[idx_u], rows_u, sem)
        cp_i = pltpu.async_copy(it_hbm.at[idx_i], rows_i, sem)
        cp_u.wait()
        cp_i.wait()

        lanes = lax.iota(jnp.int32, LANES)
        for g in range(n_groups):
            row = g * LANES + lanes
            acc = jnp.zeros((LANES,), jnp.float32)
            for d in range(4):
                col = jnp.full((LANES,), d, jnp.int32)
                u = plsc.load_gather(rows_u, [row, col])
                v = plsc.load_gather(rows_i, [row, col])
                acc = acc + u * v
            out_v[pl.ds(g * LANES, LANES)] = acc

        pltpu.sync_copy(out_v, out_hbm.at[pl.ds(base, b_per_w)])

    return run(user_idx, item_idx, user_rows8, item_rows8)


def _as_rows8(table):
    z = jnp.zeros(table.shape[:1], table.dtype)
    return jnp.stack(
        [table[:, 0], table[:, 1], table[:, 2], table[:, 3], z, z, z, z],
        axis=-1)


def kernel(user_input, item_input, user_table, item_table):
    return _score_pairs(
        user_input.astype(jnp.int32),
        item_input.astype(jnp.int32),
        _as_rows8(user_table),
        _as_rows8(item_table))


# speedup vs baseline: 20.4749x; 20.4749x over previous
import functools

import jax
import jax.numpy as jnp
from jax import lax
from jax.experimental import pallas as pl
from jax.experimental.pallas import tpu as pltpu
from jax.experimental.pallas import tpu_sc as plsc

NUM_CORES = 2
NUM_SUBCORES = 16
NUM_WORKERS = NUM_CORES * NUM_SUBCORES
LANES = 16
SLICE_W = 8


@jax.jit
def _score_pairs(idx_u_all, e_u_all, idx_i_all, e_i_all, *cols):
    batch = idx_u_all.shape[0]
    b_per_w = batch // NUM_WORKERS
    n_groups = b_per_w // LANES

    mesh = plsc.VectorSubcoreMesh(
        core_axis_name="c", subcore_axis_name="s",
        num_cores=NUM_CORES, num_subcores=NUM_SUBCORES)

    @functools.partial(
        pl.kernel,
        out_type=jax.ShapeDtypeStruct((batch,), jnp.float32),
        mesh=mesh,
        compiler_params=pltpu.CompilerParams(
            needs_layout_passes=False, use_tc_tiling_on_sc=False),
        scratch_types=[
            pltpu.VMEM((b_per_w,), jnp.int32),
            pltpu.VMEM((b_per_w,), jnp.int32),
            pltpu.VMEM((b_per_w,), jnp.int32),
            pltpu.VMEM((b_per_w,), jnp.int32),
            pltpu.VMEM((4 * b_per_w, SLICE_W), jnp.float32),
            pltpu.VMEM((4 * b_per_w, SLICE_W), jnp.float32),
            pltpu.VMEM((b_per_w,), jnp.float32),
            pltpu.SemaphoreType.DMA,
        ],
    )
    def run(iu_hbm, eu_hbm, ii_hbm, ei_hbm,
            u0h, u1h, u2h, u3h, i0h, i1h, i2h, i3h, out_hbm,
            idx_u, e_u, idx_i, e_i, sl_u, sl_i, out_v, sem):
        wid = lax.axis_index("s") * NUM_CORES + lax.axis_index("c")
        base = wid * b_per_w

        pltpu.sync_copy(iu_hbm.at[pl.ds(base, b_per_w)], idx_u)
        pltpu.sync_copy(eu_hbm.at[pl.ds(base, b_per_w)], e_u)
        pltpu.sync_copy(ii_hbm.at[pl.ds(base, b_per_w)], idx_i)
        pltpu.sync_copy(ei_hbm.at[pl.ds(base, b_per_w)], e_i)

        copies = []
        for d, col in enumerate((u0h, u1h, u2h, u3h)):
            copies.append(pltpu.async_copy(
                col.at[e_u], sl_u.at[pl.ds(d * b_per_w, b_per_w)], sem))
        for d, col in enumerate((i0h, i1h, i2h, i3h)):
            copies.append(pltpu.async_copy(
                col.at[e_i], sl_i.at[pl.ds(d * b_per_w, b_per_w)], sem))
        for cp in copies:
            cp.wait()

        lanes = lax.iota(jnp.int32, LANES)
        for g in range(n_groups):
            sl = pl.ds(g * LANES, LANES)
            j = g * LANES + lanes
            wu = jnp.bitwise_and(idx_u[sl], SLICE_W - 1)
            wi = jnp.bitwise_and(idx_i[sl], SLICE_W - 1)
            acc = jnp.zeros((LANES,), jnp.float32)
            for d in range(4):
                row = d * b_per_w + j
                u = plsc.load_gather(sl_u, [row, wu])
                v = plsc.load_gather(sl_i, [row, wi])
                acc = acc + u * v
            out_v[sl] = acc

        pltpu.sync_copy(out_v, out_hbm.at[pl.ds(base, b_per_w)])

    return run(idx_u_all, e_u_all, idx_i_all, e_i_all, *cols)


def _col8(table, d):
    return table[:, d].reshape(-1, SLICE_W)


def kernel(user_input, item_input, user_table, item_table):
    iu = user_input.astype(jnp.int32)
    ii = item_input.astype(jnp.int32)
    return _score_pairs(
        iu, iu // SLICE_W, ii, ii // SLICE_W,
        _col8(user_table, 0), _col8(user_table, 1),
        _col8(user_table, 2), _col8(user_table, 3),
        _col8(item_table, 0), _col8(item_table, 1),
        _col8(item_table, 2), _col8(item_table, 3))


# speedup vs baseline: 28.5057x; 1.3922x over previous
import functools

import jax
import jax.numpy as jnp
from jax import lax
from jax.experimental import pallas as pl
from jax.experimental.pallas import tpu as pltpu
from jax.experimental.pallas import tpu_sc as plsc

NUM_CORES = 2
NUM_SUBCORES = 16
NUM_WORKERS = NUM_CORES * NUM_SUBCORES
LANES = 16
SLICE_W = 8


@jax.jit
def _score_pairs(idx_u_all, e_u_all, idx_i_all, e_i_all, *cols):
    batch = idx_u_all.shape[0]
    b_per_w = batch // NUM_WORKERS
    n_groups = b_per_w // LANES

    mesh = plsc.VectorSubcoreMesh(
        core_axis_name="c", subcore_axis_name="s",
        num_cores=NUM_CORES, num_subcores=NUM_SUBCORES)

    @functools.partial(
        pl.kernel,
        out_type=jax.ShapeDtypeStruct((batch,), jnp.float32),
        mesh=mesh,
        compiler_params=pltpu.CompilerParams(
            needs_layout_passes=False, use_tc_tiling_on_sc=False),
        scratch_types=[
            pltpu.VMEM((b_per_w,), jnp.int32),
            pltpu.VMEM((b_per_w,), jnp.int32),
            pltpu.VMEM((b_per_w,), jnp.int32),
            pltpu.VMEM((b_per_w,), jnp.int32),
            pltpu.VMEM((4 * b_per_w, SLICE_W), jnp.float32),
            pltpu.VMEM((4 * b_per_w, SLICE_W), jnp.float32),
            pltpu.VMEM((b_per_w,), jnp.float32),
            pltpu.SemaphoreType.DMA,
        ],
    )
    def run(iu_hbm, eu_hbm, ii_hbm, ei_hbm,
            ut_hbm, it_hbm, out_hbm,
            idx_u, e_u, idx_i, e_i, sl_u, sl_i, out_v, sem):
        wid = lax.axis_index("s") * NUM_CORES + lax.axis_index("c")
        base = wid * b_per_w

        pltpu.sync_copy(iu_hbm.at[pl.ds(base, b_per_w)], idx_u)
        pltpu.sync_copy(eu_hbm.at[pl.ds(base, b_per_w)], e_u)
        pltpu.sync_copy(ii_hbm.at[pl.ds(base, b_per_w)], idx_i)
        pltpu.sync_copy(ei_hbm.at[pl.ds(base, b_per_w)], e_i)

        copies = []
        for d in range(4):
            copies.append(pltpu.async_copy(
                ut_hbm.at[d].at[e_u],
                sl_u.at[pl.ds(d * b_per_w, b_per_w)], sem))
            copies.append(pltpu.async_copy(
                it_hbm.at[d].at[e_i],
                sl_i.at[pl.ds(d * b_per_w, b_per_w)], sem))
        for cp in copies:
            cp.wait()

        lanes = lax.iota(jnp.int32, LANES)
        for g in range(n_groups):
            sl = pl.ds(g * LANES, LANES)
            j = g * LANES + lanes
            wu = jnp.bitwise_and(idx_u[sl], SLICE_W - 1)
            wi = jnp.bitwise_and(idx_i[sl], SLICE_W - 1)
            acc = jnp.zeros((LANES,), jnp.float32)
            for d in range(4):
                row = d * b_per_w + j
                u = plsc.load_gather(sl_u, [row, wu])
                v = plsc.load_gather(sl_i, [row, wi])
                acc = acc + u * v
            out_v[sl] = acc

        pltpu.sync_copy(out_v, out_hbm.at[pl.ds(base, b_per_w)])

    return run(idx_u_all, e_u_all, idx_i_all, e_i_all, *cols)


def _colsT(table):
    return table.T.reshape(4, -1, SLICE_W)


def kernel(user_input, item_input, user_table, item_table):
    iu = user_input.astype(jnp.int32)
    ii = item_input.astype(jnp.int32)
    return _score_pairs(
        iu, iu // SLICE_W, ii, ii // SLICE_W,
        _colsT(user_table), _colsT(item_table))
